# Initial kernel scaffold; baseline (speedup 1.0000x reference)
import jax
import jax.numpy as jnp
from jax.experimental import pallas as pl


def kernel(x, w1, w2, w3, g1, b1, g2, b2, g3, b3):
    raise NotImplementedError("write your pallas kernel here")


# speedup vs baseline: 1.2652x; 1.2652x over previous
import functools

import numpy as np

import jax
import jax.numpy as jnp
from jax import lax
from jax.experimental import pallas as pl
from jax.experimental.pallas import tpu as pltpu

_PAR = pltpu.CompilerParams(dimension_semantics=("parallel",))
_EPS = 1e-5
_BF = jnp.bfloat16


def _conv1_kernel(x_ref, w1_ref, y_ref, s_ref, q_ref):
    y = jnp.dot(w1_ref[...], x_ref[...].astype(_BF),
                preferred_element_type=jnp.float32)
    y_ref[...] = y.astype(_BF)
    s_ref[...] = jnp.sum(y, axis=1, keepdims=True)
    q_ref[...] = jnp.sum(y * y, axis=1, keepdims=True)


def _conv2_kernel(y1_ref, sc_ref, sh_ref, w2_ref, m_ref, y_ref, s_ref, q_ref,
                  *, offs):
    h = jnp.maximum(y1_ref[...].astype(jnp.float32) * sc_ref[...] + sh_ref[...],
                    0.0)
    hb = h.astype(_BF)
    S = hb.shape[-1]
    taps = []
    for t, off in enumerate(offs):
        shifted = hb if off == 0 else pltpu.roll(hb, (-off) % S, 1)
        taps.append(shifted * m_ref[t])
    hcat = jnp.concatenate(taps, axis=0)
    y = jnp.dot(w2_ref[...], hcat, preferred_element_type=jnp.float32)
    y_ref[...] = y.astype(_BF)
    s_ref[...] = jnp.sum(y, axis=1, keepdims=True)
    q_ref[...] = jnp.sum(y * y, axis=1, keepdims=True)


def _hgram_kernel(y2_ref, sc_ref, sh_ref, h_ref, g_ref, v_ref):
    h = jnp.maximum(y2_ref[...].astype(jnp.float32) * sc_ref[...] + sh_ref[...],
                    0.0)
    hb = h.astype(_BF)
    h_ref[...] = hb
    g_ref[...] = jnp.dot(hb, hb.T, preferred_element_type=jnp.float32)
    v_ref[...] = jnp.sum(hb.astype(jnp.float32), axis=1, keepdims=True)


def _conv3_kernel(h_ref, x_ref, w3_ref, sc_ref, sh_ref, o_ref):
    y = jnp.dot(w3_ref[...], h_ref[...], preferred_element_type=jnp.float32)
    o_ref[...] = jnp.maximum(y * sc_ref[...] + sh_ref[...] + x_ref[...], 0.0)


def _row_spec(C, S):
    return pl.BlockSpec((None, C, S), lambda b: (b, 0, 0))


def _stat_spec(C):
    return pl.BlockSpec((None, C, 1), lambda b: (b, 0, 0))


def _const_spec(shape):
    nd = len(shape)
    return pl.BlockSpec(shape, lambda b: (0,) * nd)


def _stage1(xf, w1b):
    B, Cin, S = xf.shape
    P = w1b.shape[0]
    return pl.pallas_call(
        _conv1_kernel,
        out_shape=(jax.ShapeDtypeStruct((B, P, S), _BF),
                   jax.ShapeDtypeStruct((B, P, 1), jnp.float32),
                   jax.ShapeDtypeStruct((B, P, 1), jnp.float32)),
        grid=(B,),
        in_specs=[_row_spec(Cin, S), _const_spec((P, Cin))],
        out_specs=(_row_spec(P, S), _stat_spec(P), _stat_spec(P)),
        compiler_params=_PAR,
    )(xf, w1b)


def _stage2(y1, sc, sh, w2f, masks, offs):
    B, P, S = y1.shape
    T = len(offs)
    return pl.pallas_call(
        functools.partial(_conv2_kernel, offs=tuple(offs)),
        out_shape=(jax.ShapeDtypeStruct((B, P, S), _BF),
                   jax.ShapeDtypeStruct((B, P, 1), jnp.float32),
                   jax.ShapeDtypeStruct((B, P, 1), jnp.float32)),
        grid=(B,),
        in_specs=[_row_spec(P, S),
                  _const_spec((P, 1)), _const_spec((P, 1)),
                  _const_spec((P, T * P)), _const_spec((T, 1, S))],
        out_specs=(_row_spec(P, S), _stat_spec(P), _stat_spec(P)),
        compiler_params=_PAR,
    )(y1, sc, sh, w2f, masks)


def _stage3(y2, sc, sh):
    B, P, S = y2.shape
    return pl.pallas_call(
        _hgram_kernel,
        out_shape=(jax.ShapeDtypeStruct((B, P, S), _BF),
                   jax.ShapeDtypeStruct((B, P, P), jnp.float32),
                   jax.ShapeDtypeStruct((B, P, 1), jnp.float32)),
        grid=(B,),
        in_specs=[_row_spec(P, S),
                  _const_spec((P, 1)), _const_spec((P, 1))],
        out_specs=(_row_spec(P, S),
                   pl.BlockSpec((None, P, P), lambda b: (b, 0, 0)),
                   _stat_spec(P)),
        compiler_params=_PAR,
    )(y2, sc, sh)


def _stage4(h2, xf, w3b, sc, sh):
    B, P, S = h2.shape
    Pe = w3b.shape[0]
    return pl.pallas_call(
        _conv3_kernel,
        out_shape=jax.ShapeDtypeStruct((B, Pe, S), jnp.float32),
        grid=(B,),
        in_specs=[_row_spec(P, S), _row_spec(Pe, S),
                  _const_spec((Pe, P)),
                  _const_spec((Pe, 1)), _const_spec((Pe, 1))],
        out_specs=_row_spec(Pe, S),
        compiler_params=_PAR,
    )(h2, xf, w3b, sc, sh)


def _bn_fold(ssum, ssq, count, gamma, beta):
    tot = jnp.sum(ssum[:, :, 0], axis=0)
    tot2 = jnp.sum(ssq[:, :, 0], axis=0)
    mean = tot / count
    var = tot2 / count - mean * mean
    scale = gamma * lax.rsqrt(var + _EPS)
    shift = beta - mean * scale
    return scale[:, None], shift[:, None]


def _tap_tables(H, W, D):
    hh, ww, dd = np.meshgrid(np.arange(H), np.arange(W), np.arange(D),
                             indexing="ij")
    offs, masks = [], []
    for ki in range(3):
        for kj in range(3):
            for kk in range(3):
                dh, dw, dz = ki - 1, kj - 1, kk - 1
                valid = ((hh + dh >= 0) & (hh + dh < H) &
                         (ww + dw >= 0) & (ww + dw < W) &
                         (dd + dz >= 0) & (dd + dz < D))
                offs.append(dh * W * D + dw * D + dz)
                masks.append(valid.reshape(-1))
    masks = jnp.asarray(np.stack(masks)[:, None, :], dtype=_BF)
    return tuple(offs), masks


def kernel(x, w1, w2, w3, g1, b1, g2, b2, g3, b3):
    B, Cin, H, W, D = x.shape
    S = H * W * D
    P = w2.shape[0]
    count = B * S

    xf = x.reshape(B, Cin, S)
    w1b = w1.astype(_BF)
    w2f = w2.transpose(0, 2, 3, 4, 1).reshape(P, 27 * P).astype(_BF)
    w3b = w3.astype(_BF)
    offs, masks = _tap_tables(H, W, D)

    y1, s1, q1 = _stage1(xf, w1b)
    sc1, sh1 = _bn_fold(s1, q1, count, g1, b1)

    y2, s2, q2 = _stage2(y1, sc1, sh1, w2f, masks, offs)
    sc2, sh2 = _bn_fold(s2, q2, count, g2, b2)

    h2, g, v = _stage3(y2, sc2, sh2)

    w3f = w3b.astype(jnp.float32)
    gtot = jnp.sum(g, axis=0)
    hsum = jnp.sum(v[:, :, 0], axis=0)
    s3 = w3f @ hsum
    q3 = jnp.sum((w3f @ gtot) * w3f, axis=1)
    mean3 = s3 / count
    var3 = q3 / count - mean3 * mean3
    sc3 = g3 * lax.rsqrt(var3 + _EPS)
    sh3 = b3 - mean3 * sc3

    out = _stage4(h2, xf, w3b, sc3[:, None], sh3[:, None])
    return out.reshape(B, Cin, H, W, D)


# speedup vs baseline: 2.2570x; 1.7840x over previous
import functools

import numpy as np

import jax
import jax.numpy as jnp
from jax import lax
from jax.experimental import pallas as pl
from jax.experimental.pallas import tpu as pltpu

_PAR = pltpu.CompilerParams(dimension_semantics=("parallel",))
_EPS = 1e-5
_BF = jnp.bfloat16


def _conv1_kernel(xt_ref, w1_ref, y_ref, s_ref, q_ref):
    y = lax.dot_general(w1_ref[...], xt_ref[...].astype(_BF),
                        (((1,), (1,)), ((), ())),
                        preferred_element_type=jnp.float32)
    y_ref[...] = y.astype(_BF)
    s_ref[...] = jnp.sum(y, axis=1, keepdims=True)
    q_ref[...] = jnp.sum(y * y, axis=1, keepdims=True)


def _conv2_kernel(y1_ref, sc_ref, sh_ref, w2_ref, m_ref, y_ref, s_ref, q_ref,
                  *, offs):
    h = jnp.maximum(y1_ref[...].astype(jnp.float32) * sc_ref[...] + sh_ref[...],
                    0.0)
    hb = h.astype(_BF)
    S = hb.shape[-1]
    taps = []
    for t, off in enumerate(offs):
        shifted = hb if off == 0 else pltpu.roll(hb, (-off) % S, 1)
        taps.append(shifted * m_ref[t])
    hcat = jnp.concatenate(taps, axis=0)
    y = jnp.dot(w2_ref[...], hcat, preferred_element_type=jnp.float32)
    y_ref[...] = y.astype(_BF)
    s_ref[...] = jnp.sum(y, axis=1, keepdims=True)
    q_ref[...] = jnp.sum(y * y, axis=1, keepdims=True)


def _hgram_kernel(y2_ref, sc_ref, sh_ref, h_ref, g_ref, v_ref):
    h = jnp.maximum(y2_ref[...].astype(jnp.float32) * sc_ref[...] + sh_ref[...],
                    0.0)
    hb = h.astype(_BF)
    h_ref[...] = hb
    g_ref[...] = jnp.dot(hb, hb.T, preferred_element_type=jnp.float32)
    v_ref[...] = jnp.sum(hb.astype(jnp.float32), axis=1, keepdims=True)


def _conv3_kernel(h_ref, xt_ref, w3_ref, sc_ref, sh_ref, o_ref):
    y = lax.dot_general(h_ref[...], w3_ref[...],
                        (((0,), (1,)), ((), ())),
                        preferred_element_type=jnp.float32)
    o_ref[...] = jnp.maximum(y * sc_ref[...] + sh_ref[...] + xt_ref[...], 0.0)


def _row_spec(C, S):
    return pl.BlockSpec((None, C, S), lambda b: (b, 0, 0))


def _stat_spec(C):
    return pl.BlockSpec((None, C, 1), lambda b: (b, 0, 0))


def _const_spec(shape):
    nd = len(shape)
    return pl.BlockSpec(shape, lambda b: (0,) * nd)


def _stage1(xt, w1b):
    B, S, Cin = xt.shape
    P = w1b.shape[0]
    return pl.pallas_call(
        _conv1_kernel,
        out_shape=(jax.ShapeDtypeStruct((B, P, S), _BF),
                   jax.ShapeDtypeStruct((B, P, 1), jnp.float32),
                   jax.ShapeDtypeStruct((B, P, 1), jnp.float32)),
        grid=(B,),
        in_specs=[_row_spec(S, Cin), _const_spec((P, Cin))],
        out_specs=(_row_spec(P, S), _stat_spec(P), _stat_spec(P)),
        compiler_params=_PAR,
    )(xt, w1b)


def _stage2(y1, sc, sh, w2f, masks, offs):
    B, P, S = y1.shape
    T = len(offs)
    return pl.pallas_call(
        functools.partial(_conv2_kernel, offs=tuple(offs)),
        out_shape=(jax.ShapeDtypeStruct((B, P, S), _BF),
                   jax.ShapeDtypeStruct((B, P, 1), jnp.float32),
                   jax.ShapeDtypeStruct((B, P, 1), jnp.float32)),
        grid=(B,),
        in_specs=[_row_spec(P, S),
                  _const_spec((P, 1)), _const_spec((P, 1)),
                  _const_spec((P, T * P)), _const_spec((T, 1, S))],
        out_specs=(_row_spec(P, S), _stat_spec(P), _stat_spec(P)),
        compiler_params=_PAR,
    )(y1, sc, sh, w2f, masks)


def _stage3(y2, sc, sh):
    B, P, S = y2.shape
    return pl.pallas_call(
        _hgram_kernel,
        out_shape=(jax.ShapeDtypeStruct((B, P, S), _BF),
                   jax.ShapeDtypeStruct((B, P, P), jnp.float32),
                   jax.ShapeDtypeStruct((B, P, 1), jnp.float32)),
        grid=(B,),
        in_specs=[_row_spec(P, S),
                  _const_spec((P, 1)), _const_spec((P, 1))],
        out_specs=(_row_spec(P, S),
                   pl.BlockSpec((None, P, P), lambda b: (b, 0, 0)),
                   _stat_spec(P)),
        compiler_params=_PAR,
    )(y2, sc, sh)


def _stage4(h2, xt, w3b, sc, sh):
    B, P, S = h2.shape
    Pe = w3b.shape[0]
    return pl.pallas_call(
        _conv3_kernel,
        out_shape=jax.ShapeDtypeStruct((B, S, Pe), jnp.float32),
        grid=(B,),
        in_specs=[_row_spec(P, S), _row_spec(S, Pe),
                  _const_spec((Pe, P)),
                  _const_spec((1, Pe)), _const_spec((1, Pe))],
        out_specs=_row_spec(S, Pe),
        compiler_params=_PAR,
    )(h2, xt, w3b, sc, sh)


def _bn_fold(ssum, ssq, count, gamma, beta):
    tot = jnp.sum(ssum[:, :, 0], axis=0)
    tot2 = jnp.sum(ssq[:, :, 0], axis=0)
    mean = tot / count
    var = tot2 / count - mean * mean
    scale = gamma * lax.rsqrt(var + _EPS)
    shift = beta - mean * scale
    return scale[:, None], shift[:, None]


def _tap_tables(H, W, D):
    hh, ww, dd = np.meshgrid(np.arange(H), np.arange(W), np.arange(D),
                             indexing="ij")
    offs, masks = [], []
    for ki in range(3):
        for kj in range(3):
            for kk in range(3):
                dh, dw, dz = ki - 1, kj - 1, kk - 1
                valid = ((hh + dh >= 0) & (hh + dh < H) &
                         (ww + dw >= 0) & (ww + dw < W) &
                         (dd + dz >= 0) & (dd + dz < D))
                offs.append(dh * W * D + dw * D + dz)
                masks.append(valid.reshape(-1))
    masks = jnp.asarray(np.stack(masks)[:, None, :], dtype=_BF)
    return tuple(offs), masks


def kernel(x, w1, w2, w3, g1, b1, g2, b2, g3, b3):
    B, Cin, H, W, D = x.shape
    S = H * W * D
    P = w2.shape[0]
    count = B * S

    xt = x.transpose(0, 2, 3, 4, 1).reshape(B, S, Cin)
    w1b = w1.astype(_BF)
    w2f = w2.transpose(0, 2, 3, 4, 1).reshape(P, 27 * P).astype(_BF)
    w3b = w3.astype(_BF)
    offs, masks = _tap_tables(H, W, D)

    y1, s1, q1 = _stage1(xt, w1b)
    sc1, sh1 = _bn_fold(s1, q1, count, g1, b1)

    y2, s2, q2 = _stage2(y1, sc1, sh1, w2f, masks, offs)
    sc2, sh2 = _bn_fold(s2, q2, count, g2, b2)

    h2, g, v = _stage3(y2, sc2, sh2)

    w3f = w3b.astype(jnp.float32)
    gtot = jnp.sum(g, axis=0)
    hsum = jnp.sum(v[:, :, 0], axis=0)
    s3 = w3f @ hsum
    q3 = jnp.sum((w3f @ gtot) * w3f, axis=1)
    mean3 = s3 / count
    var3 = q3 / count - mean3 * mean3
    sc3 = g3 * lax.rsqrt(var3 + _EPS)
    sh3 = b3 - mean3 * sc3

    out = _stage4(h2, xt, w3b, sc3[None, :], sh3[None, :])
    return out.reshape(B, H, W, D, Cin).transpose(0, 4, 1, 2, 3)


# speedup vs baseline: 2.5336x; 1.1225x over previous
import functools

import numpy as np

import jax
import jax.numpy as jnp
from jax import lax
from jax.experimental import pallas as pl
from jax.experimental.pallas import tpu as pltpu

_PAR = pltpu.CompilerParams(dimension_semantics=("parallel",))
_EPS = 1e-5
_BF = jnp.bfloat16


def _conv1_kernel(xt_ref, w1_ref, y_ref, s_ref, q_ref):
    y = lax.dot_general(w1_ref[...], xt_ref[...].astype(_BF),
                        (((1,), (1,)), ((), ())),
                        preferred_element_type=jnp.float32)
    y_ref[...] = y.astype(_BF)
    s_ref[...] = jnp.sum(y, axis=1, keepdims=True)
    q_ref[...] = jnp.sum(y * y, axis=1, keepdims=True)


def _taps_for(hb, m_ref, offs, S):
    pres = {}
    for dh, inner in offs:
        if inner not in pres:
            pres[inner] = hb if inner == 0 else pltpu.roll(hb, (-inner) % S, 1)
    taps = []
    for t, (dh, inner) in enumerate(offs):
        p = pres[inner]
        shifted = p if dh == 0 else pltpu.roll(p, (-dh) % S, 1)
        taps.append(shifted * m_ref[t])
    return taps


def _conv2_kernel(y1_ref, sc_ref, sh_ref, w2_ref, m_ref, y_ref, s_ref, q_ref,
                  *, offs, half_s):
    h = jnp.maximum(y1_ref[...].astype(jnp.float32) * sc_ref[...] + sh_ref[...],
                    0.0)
    hb = h.astype(_BF)
    acc_s = jnp.zeros((hb.shape[0], 1), jnp.float32)
    acc_q = jnp.zeros((hb.shape[0], 1), jnp.float32)
    for half in range(2):
        hh = hb[:, half * half_s:(half + 1) * half_s]
        taps = _taps_for(hh, m_ref, offs, half_s)
        hcat = jnp.concatenate(taps, axis=0)
        y = jnp.dot(w2_ref[...], hcat, preferred_element_type=jnp.float32)
        y_ref[:, half * half_s:(half + 1) * half_s] = y.astype(_BF)
        acc_s = acc_s + jnp.sum(y, axis=1, keepdims=True)
        acc_q = acc_q + jnp.sum(y * y, axis=1, keepdims=True)
    s_ref[...] = acc_s
    q_ref[...] = acc_q


def _hgram_kernel(y2_ref, sc_ref, sh_ref, g_ref, v_ref):
    h = jnp.maximum(y2_ref[...].astype(jnp.float32) * sc_ref[...] + sh_ref[...],
                    0.0)
    hb = h.astype(_BF)
    g_ref[...] = lax.dot_general(hb, hb, (((1,), (1,)), ((), ())),
                                 preferred_element_type=jnp.float32)
    v_ref[...] = jnp.sum(hb.astype(jnp.float32), axis=1, keepdims=True)


def _conv3_kernel(y2_ref, xt_ref, w3_ref, sc2_ref, sh2_ref, sc_ref, sh_ref,
                  o_ref, *, half_s):
    for half in range(2):
        y2h = y2_ref[:, half * half_s:(half + 1) * half_s]
        h = jnp.maximum(
            y2h.astype(jnp.float32) * sc2_ref[...] + sh2_ref[...], 0.0)
        hb = h.astype(_BF)
        y = lax.dot_general(hb, w3_ref[...], (((0,), (1,)), ((), ())),
                            preferred_element_type=jnp.float32)
        rows = pl.ds(half * half_s, half_s)
        o_ref[rows, :] = jnp.maximum(
            y * sc_ref[...] + sh_ref[...] + xt_ref[rows, :], 0.0)


def _row_spec(C, S):
    return pl.BlockSpec((None, C, S), lambda b: (b, 0, 0))


def _stat_spec(C):
    return pl.BlockSpec((None, C, 1), lambda b: (b, 0, 0))


def _const_spec(shape):
    nd = len(shape)
    return pl.BlockSpec(shape, lambda b: (0,) * nd)


def _stage1(xtp, w1b):
    BP, S2, Cin = xtp.shape
    P = w1b.shape[0]
    return pl.pallas_call(
        _conv1_kernel,
        out_shape=(jax.ShapeDtypeStruct((BP, P, S2), _BF),
                   jax.ShapeDtypeStruct((BP, P, 1), jnp.float32),
                   jax.ShapeDtypeStruct((BP, P, 1), jnp.float32)),
        grid=(BP,),
        in_specs=[_row_spec(S2, Cin), _const_spec((P, Cin))],
        out_specs=(_row_spec(P, S2), _stat_spec(P), _stat_spec(P)),
        compiler_params=_PAR,
    )(xtp, w1b)


def _stage2(y1, sc, sh, w2f, masks, offs):
    BP, P, S2 = y1.shape
    T = masks.shape[0]
    return pl.pallas_call(
        functools.partial(_conv2_kernel, offs=offs, half_s=S2 // 2),
        out_shape=(jax.ShapeDtypeStruct((BP, P, S2), _BF),
                   jax.ShapeDtypeStruct((BP, P, 1), jnp.float32),
                   jax.ShapeDtypeStruct((BP, P, 1), jnp.float32)),
        grid=(BP,),
        in_specs=[_row_spec(P, S2),
                  _const_spec((P, 1)), _const_spec((P, 1)),
                  _const_spec((P, T * P)), _const_spec((T, 1, S2 // 2))],
        out_specs=(_row_spec(P, S2), _stat_spec(P), _stat_spec(P)),
        compiler_params=_PAR,
    )(y1, sc, sh, w2f, masks)


def _stage3(y2, sc, sh):
    BP, P, S2 = y2.shape
    return pl.pallas_call(
        _hgram_kernel,
        out_shape=(jax.ShapeDtypeStruct((BP, P, P), jnp.float32),
                   jax.ShapeDtypeStruct((BP, P, 1), jnp.float32)),
        grid=(BP,),
        in_specs=[_row_spec(P, S2),
                  _const_spec((P, 1)), _const_spec((P, 1))],
        out_specs=(pl.BlockSpec((None, P, P), lambda b: (b, 0, 0)),
                   _stat_spec(P)),
        compiler_params=_PAR,
    )(y2, sc, sh)


def _stage4(y2, xtp, w3b, sc2, sh2, sc, sh):
    BP, P, S2 = y2.shape
    Pe = w3b.shape[0]
    return pl.pallas_call(
        functools.partial(_conv3_kernel, half_s=S2 // 2),
        out_shape=jax.ShapeDtypeStruct((BP, S2, Pe), jnp.float32),
        grid=(BP,),
        in_specs=[_row_spec(P, S2), _row_spec(S2, Pe),
                  _const_spec((Pe, P)),
                  _const_spec((P, 1)), _const_spec((P, 1)),
                  _const_spec((1, Pe)), _const_spec((1, Pe))],
        out_specs=_row_spec(S2, Pe),
        compiler_params=_PAR,
    )(y2, xtp, w3b, sc2, sh2, sc, sh)


def _bn_fold(ssum, ssq, count, gamma, beta):
    tot = jnp.sum(ssum[:, :, 0], axis=0)
    tot2 = jnp.sum(ssq[:, :, 0], axis=0)
    mean = tot / count
    var = tot2 / count - mean * mean
    scale = gamma * lax.rsqrt(var + _EPS)
    shift = beta - mean * scale
    return scale[:, None], shift[:, None]


def _tap_tables(H, W, D):
    hh, ww, dd = np.meshgrid(np.arange(H), np.arange(W), np.arange(D),
                             indexing="ij")
    offs, masks = [], []
    for ki in range(3):
        for kj in range(3):
            for kk in range(3):
                dh, dw, dz = ki - 1, kj - 1, kk - 1
                valid = ((hh + dh >= 0) & (hh + dh < H) &
                         (ww + dw >= 0) & (ww + dw < W) &
                         (dd + dz >= 0) & (dd + dz < D))
                offs.append((dh * W * D, dw * D + dz))
                masks.append(valid.reshape(-1))
    masks = jnp.asarray(np.stack(masks)[:, None, :], dtype=_BF)
    return tuple(offs), masks


def kernel(x, w1, w2, w3, g1, b1, g2, b2, g3, b3):
    B, Cin, H, W, D = x.shape
    S = H * W * D
    P = w2.shape[0]
    count = B * S

    xtp = x.transpose(0, 2, 3, 4, 1).reshape(B // 2, 2 * S, Cin)
    w1b = w1.astype(_BF)
    w2f = w2.astype(_BF).transpose(0, 2, 3, 4, 1).reshape(P, 27 * P)
    w3b = w3.astype(_BF)
    offs, masks = _tap_tables(H, W, D)

    y1, s1, q1 = _stage1(xtp, w1b)
    sc1, sh1 = _bn_fold(s1, q1, count, g1, b1)

    y2, s2, q2 = _stage2(y1, sc1, sh1, w2f, masks, offs)
    sc2, sh2 = _bn_fold(s2, q2, count, g2, b2)

    g, v = _stage3(y2, sc2, sh2)

    w3f = w3b.astype(jnp.float32)
    gtot = jnp.sum(g, axis=0)
    hsum = jnp.sum(v[:, :, 0], axis=0)
    s3 = w3f @ hsum
    q3 = jnp.sum((w3f @ gtot) * w3f, axis=1)
    mean3 = s3 / count
    var3 = q3 / count - mean3 * mean3
    sc3 = g3 * lax.rsqrt(var3 + _EPS)
    sh3 = b3 - mean3 * sc3

    out = _stage4(y2, xtp, w3b, sc2, sh2, sc3[None, :], sh3[None, :])
    return out.reshape(B, H, W, D, Cin).transpose(0, 4, 1, 2, 3)


# speedup vs baseline: 2.6045x; 1.0280x over previous
import functools

import numpy as np

import jax
import jax.numpy as jnp
from jax import lax
from jax.experimental import pallas as pl
from jax.experimental.pallas import tpu as pltpu

_SEQ = pltpu.CompilerParams(dimension_semantics=("arbitrary",))
_EPS = 1e-5
_BF = jnp.bfloat16


def _fold(s_ref, q_ref, gb_ref, count):
    mean = s_ref[...] / count
    var = q_ref[...] / count - mean * mean
    gbt = jnp.transpose(gb_ref[...])
    sc = gbt[:, 0:1] * lax.rsqrt(var + _EPS)
    sh = gbt[:, 1:2] - mean * sc
    return sc, sh


def _acc_stats(step, s_ref, q_ref, s_val, q_val):
    @pl.when(step == 0)
    def _():
        s_ref[...] = jnp.zeros_like(s_ref)
        q_ref[...] = jnp.zeros_like(q_ref)
    s_ref[...] += s_val
    q_ref[...] += q_val


def _conv1_kernel(xt_ref, w1_ref, y_ref, s_ref, q_ref):
    y = lax.dot_general(w1_ref[...], xt_ref[...].astype(_BF),
                        (((1,), (1,)), ((), ())),
                        preferred_element_type=jnp.float32)
    y_ref[...] = y.astype(_BF)
    _acc_stats(pl.program_id(0), s_ref, q_ref,
               jnp.sum(y, axis=1, keepdims=True),
               jnp.sum(y * y, axis=1, keepdims=True))


def _taps_for(hb, m_ref, offs, S):
    pres = {}
    for dh, inner in offs:
        if inner not in pres:
            pres[inner] = hb if inner == 0 else pltpu.roll(hb, (-inner) % S, 1)
    taps = []
    for t, (dh, inner) in enumerate(offs):
        p = pres[inner]
        shifted = p if dh == 0 else pltpu.roll(p, (-dh) % S, 1)
        taps.append(shifted * m_ref[t])
    return taps


def _conv2_kernel(y1_ref, s1_ref, q1_ref, gb_ref, w2_ref, m_ref,
                  y_ref, s_ref, q_ref, *, offs, half_s, count):
    sc, sh = _fold(s1_ref, q1_ref, gb_ref, count)
    h = jnp.maximum(y1_ref[...].astype(jnp.float32) * sc + sh, 0.0)
    hb = h.astype(_BF)
    acc_s = jnp.zeros((hb.shape[0], 1), jnp.float32)
    acc_q = jnp.zeros((hb.shape[0], 1), jnp.float32)
    for half in range(2):
        hh = hb[:, half * half_s:(half + 1) * half_s]
        taps = _taps_for(hh, m_ref, offs, half_s)
        hcat = jnp.concatenate(taps, axis=0)
        y = jnp.dot(w2_ref[...], hcat, preferred_element_type=jnp.float32)
        y_ref[:, half * half_s:(half + 1) * half_s] = y.astype(_BF)
        acc_s = acc_s + jnp.sum(y, axis=1, keepdims=True)
        acc_q = acc_q + jnp.sum(y * y, axis=1, keepdims=True)
    _acc_stats(pl.program_id(0), s_ref, q_ref, acc_s, acc_q)


def _hgram_kernel(y2_ref, s2_ref, q2_ref, gb_ref, g_ref, v_ref, *, count):
    sc, sh = _fold(s2_ref, q2_ref, gb_ref, count)
    h = jnp.maximum(y2_ref[...].astype(jnp.float32) * sc + sh, 0.0)
    hb = h.astype(_BF)
    g = lax.dot_general(hb, hb, (((1,), (1,)), ((), ())),
                        preferred_element_type=jnp.float32)
    v = jnp.sum(hb.astype(jnp.float32), axis=1, keepdims=True)
    step = pl.program_id(0)

    @pl.when(step == 0)
    def _():
        g_ref[...] = jnp.zeros_like(g_ref)
        v_ref[...] = jnp.zeros_like(v_ref)
    g_ref[...] += g
    v_ref[...] += v


def _conv3_kernel(y2_ref, xt_ref, w3_ref, s2_ref, q2_ref, gb_ref,
                  sc_ref, sh_ref, o_ref, *, half_s, count):
    sc2, sh2 = _fold(s2_ref, q2_ref, gb_ref, count)
    for half in range(2):
        y2h = y2_ref[:, half * half_s:(half + 1) * half_s]
        h = jnp.maximum(y2h.astype(jnp.float32) * sc2 + sh2, 0.0)
        hb = h.astype(_BF)
        y = lax.dot_general(hb, w3_ref[...], (((0,), (1,)), ((), ())),
                            preferred_element_type=jnp.float32)
        rows = pl.ds(half * half_s, half_s)
        o_ref[rows, :] = jnp.maximum(
            y * sc_ref[...] + sh_ref[...] + xt_ref[rows, :], 0.0)


def _row_spec(C, S):
    return pl.BlockSpec((None, C, S), lambda b: (b, 0, 0))


def _acc_spec(shape):
    nd = len(shape)
    return pl.BlockSpec(shape, lambda b: (0,) * nd)


_const_spec = _acc_spec


def _stage1(xtp, w1b):
    BP, S2, Cin = xtp.shape
    P = w1b.shape[0]
    return pl.pallas_call(
        _conv1_kernel,
        out_shape=(jax.ShapeDtypeStruct((BP, P, S2), _BF),
                   jax.ShapeDtypeStruct((P, 1), jnp.float32),
                   jax.ShapeDtypeStruct((P, 1), jnp.float32)),
        grid=(BP,),
        in_specs=[_row_spec(S2, Cin), _const_spec((P, Cin))],
        out_specs=(_row_spec(P, S2), _acc_spec((P, 1)), _acc_spec((P, 1))),
        compiler_params=_SEQ,
    )(xtp, w1b)


def _stage2(y1, s1, q1, gb1, w2f, masks, offs, count):
    BP, P, S2 = y1.shape
    T = masks.shape[0]
    return pl.pallas_call(
        functools.partial(_conv2_kernel, offs=offs, half_s=S2 // 2,
                          count=count),
        out_shape=(jax.ShapeDtypeStruct((BP, P, S2), _BF),
                   jax.ShapeDtypeStruct((P, 1), jnp.float32),
                   jax.ShapeDtypeStruct((P, 1), jnp.float32)),
        grid=(BP,),
        in_specs=[_row_spec(P, S2),
                  _const_spec((P, 1)), _const_spec((P, 1)),
                  _const_spec((2, P)),
                  _const_spec((P, T * P)), _const_spec((T, 1, S2 // 2))],
        out_specs=(_row_spec(P, S2), _acc_spec((P, 1)), _acc_spec((P, 1))),
        compiler_params=_SEQ,
    )(y1, s1, q1, gb1, w2f, masks)


def _stage3(y2, s2, q2, gb2, count):
    BP, P, S2 = y2.shape
    return pl.pallas_call(
        functools.partial(_hgram_kernel, count=count),
        out_shape=(jax.ShapeDtypeStruct((P, P), jnp.float32),
                   jax.ShapeDtypeStruct((P, 1), jnp.float32)),
        grid=(BP,),
        in_specs=[_row_spec(P, S2),
                  _const_spec((P, 1)), _const_spec((P, 1)),
                  _const_spec((2, P))],
        out_specs=(_acc_spec((P, P)), _acc_spec((P, 1))),
        compiler_params=_SEQ,
    )(y2, s2, q2, gb2)


def _stage4(y2, xtp, w3b, s2, q2, gb2, sc3, sh3, count):
    BP, P, S2 = y2.shape
    Pe = w3b.shape[0]
    return pl.pallas_call(
        functools.partial(_conv3_kernel, half_s=S2 // 2, count=count),
        out_shape=jax.ShapeDtypeStruct((BP, S2, Pe), jnp.float32),
        grid=(BP,),
        in_specs=[_row_spec(P, S2), _row_spec(S2, Pe),
                  _const_spec((Pe, P)),
                  _const_spec((P, 1)), _const_spec((P, 1)),
                  _const_spec((2, P)),
                  _const_spec((1, Pe)), _const_spec((1, Pe))],
        out_specs=_row_spec(S2, Pe),
        compiler_params=_SEQ,
    )(y2, xtp, w3b, s2, q2, gb2, sc3, sh3)


def _tap_tables(H, W, D):
    hh, ww, dd = np.meshgrid(np.arange(H), np.arange(W), np.arange(D),
                             indexing="ij")
    offs, masks = [], []
    for ki in range(3):
        for kj in range(3):
            for kk in range(3):
                dh, dw, dz = ki - 1, kj - 1, kk - 1
                valid = ((hh + dh >= 0) & (hh + dh < H) &
                         (ww + dw >= 0) & (ww + dw < W) &
                         (dd + dz >= 0) & (dd + dz < D))
                offs.append((dh * W * D, dw * D + dz))
                masks.append(valid.reshape(-1))
    masks = jnp.asarray(np.stack(masks)[:, None, :], dtype=_BF)
    return tuple(offs), masks


def kernel(x, w1, w2, w3, g1, b1, g2, b2, g3, b3):
    B, Cin, H, W, D = x.shape
    S = H * W * D
    P = w2.shape[0]
    count = float(B * S)

    xtp = x.transpose(0, 2, 3, 4, 1).reshape(B // 2, 2 * S, Cin)
    w1b = w1.astype(_BF)
    w2f = w2.astype(_BF).transpose(0, 2, 3, 4, 1).reshape(P, 27 * P)
    w3b = w3.astype(_BF)
    gb1 = jnp.stack([g1, b1])
    gb2 = jnp.stack([g2, b2])
    offs, masks = _tap_tables(H, W, D)

    y1, s1, q1 = _stage1(xtp, w1b)
    y2, s2, q2 = _stage2(y1, s1, q1, gb1, w2f, masks, offs, count)
    g, v = _stage3(y2, s2, q2, gb2, count)

    w3f = w3b.astype(jnp.float32)
    s3 = w3f @ v[:, 0]
    q3 = jnp.sum((w3f @ g) * w3f, axis=1)
    mean3 = s3 / count
    var3 = q3 / count - mean3 * mean3
    sc3 = g3 * lax.rsqrt(var3 + _EPS)
    sh3 = b3 - mean3 * sc3

    out = _stage4(y2, xtp, w3b, s2, q2, gb2, sc3[None, :], sh3[None, :],
                  count)
    return out.reshape(B, H, W, D, Cin).transpose(0, 4, 1, 2, 3)
